# baseline (speedup 1.0000x reference)
import functools

import jax
import jax.numpy as jnp
from jax import lax
from jax.experimental import pallas as pl
from jax.experimental.pallas import tpu as pltpu
from jax.experimental.pallas import tpu_sc as plsc

NC = 2
NS = 16
L = 16
BATCH = 16384
BPW = BATCH // (NC * NS)
D_CONST = 1.702


def _irt_body(uidx_hbm, aidx_hbm, bidx_hbm, cidx_hbm, w_hbm, out_hbm,
              uidx_v, aidx_v, bidx_v, cidx_v,
              th_v, a_v, b_v, c_v, out_v, sem):
  wid = lax.axis_index("s") * NC + lax.axis_index("c")
  base = wid * BPW

  pltpu.sync_copy(uidx_hbm.at[pl.ds(base, BPW)], uidx_v)
  pltpu.sync_copy(aidx_hbm.at[pl.ds(base, BPW)], aidx_v)
  pltpu.sync_copy(bidx_hbm.at[pl.ds(base, BPW)], bidx_v)
  pltpu.sync_copy(cidx_hbm.at[pl.ds(base, BPW)], cidx_v)

  c1 = pltpu.async_copy(w_hbm.at[uidx_v], th_v, sem)
  c2 = pltpu.async_copy(w_hbm.at[aidx_v], a_v, sem)
  c3 = pltpu.async_copy(w_hbm.at[bidx_v], b_v, sem)
  c4 = pltpu.async_copy(w_hbm.at[cidx_v], c_v, sem)
  c1.wait()
  c2.wait()
  c3.wait()
  c4.wait()

  def step(i, _):
    sl = pl.ds(i * L, L)
    th = th_v[sl]
    a = a_v[sl]
    b = b_v[sl]
    c = c_v[sl]
    a_s = 1.0 / (1.0 + jnp.exp(-a))
    c_s = 1.0 / (1.0 + jnp.exp(-c))
    z = 1.0 / (1.0 + jnp.exp(-D_CONST * a_s * (th - b)))
    out_v[sl] = c_s + (1.0 - c_s) * z
    return 0

  lax.fori_loop(0, BPW // L, step, 0, unroll=4)

  pltpu.sync_copy(out_v, out_hbm.at[pl.ds(base, BPW)])


@jax.jit
def _irt_sc(user, item, theta_w, a_w, b_w, c_w):
  w = jnp.reshape(
      jnp.squeeze(jnp.stack([theta_w, a_w, b_w, c_w], axis=1), -1), (-1,))
  w = lax.optimization_barrier(w)
  uidx = user * 4
  aidx = item * 4 + 1
  bidx = item * 4 + 2
  cidx = item * 4 + 3
  mesh = plsc.VectorSubcoreMesh(
      core_axis_name="c", subcore_axis_name="s",
      num_cores=NC, num_subcores=NS)
  fn = pl.kernel(
      _irt_body,
      out_type=jax.ShapeDtypeStruct((BATCH,), jnp.float32),
      mesh=mesh,
      scratch_types=[
          pltpu.VMEM((BPW,), jnp.int32),
          pltpu.VMEM((BPW,), jnp.int32),
          pltpu.VMEM((BPW,), jnp.int32),
          pltpu.VMEM((BPW,), jnp.int32),
          pltpu.VMEM((BPW,), jnp.float32),
          pltpu.VMEM((BPW,), jnp.float32),
          pltpu.VMEM((BPW,), jnp.float32),
          pltpu.VMEM((BPW,), jnp.float32),
          pltpu.VMEM((BPW,), jnp.float32),
          pltpu.SemaphoreType.DMA,
      ],
  )
  return fn(uidx, aidx, bidx, cidx, w)


def kernel(user, item, theta_w, a_w, b_w, c_w):
  return _irt_sc(user, item, theta_w, a_w, b_w, c_w)


# speedup vs baseline: 4.4126x; 4.4126x over previous
import functools

import jax
import jax.numpy as jnp
from jax import lax
from jax.experimental import pallas as pl
from jax.experimental.pallas import tpu as pltpu
from jax.experimental.pallas import tpu_sc as plsc

NC = 2
NS = 16
L = 16
BATCH = 16384
BPW = BATCH // (NC * NS)
D_CONST = 1.702


def _irt_body(uidx_hbm, aidx_hbm, bidx_hbm, cidx_hbm, w_hbm, out_hbm,
              uidx_v, aidx_v, bidx_v, cidx_v,
              th_v, a_v, b_v, c_v, out_v, sem):
  wid = lax.axis_index("s") * NC + lax.axis_index("c")
  base = wid * BPW

  pltpu.sync_copy(uidx_hbm.at[pl.ds(base, BPW)], uidx_v)
  pltpu.sync_copy(aidx_hbm.at[pl.ds(base, BPW)], aidx_v)
  pltpu.sync_copy(bidx_hbm.at[pl.ds(base, BPW)], bidx_v)
  pltpu.sync_copy(cidx_hbm.at[pl.ds(base, BPW)], cidx_v)

  c1 = pltpu.async_copy(w_hbm.at[uidx_v], th_v, sem)
  c2 = pltpu.async_copy(w_hbm.at[aidx_v], a_v, sem)
  c3 = pltpu.async_copy(w_hbm.at[bidx_v], b_v, sem)
  c4 = pltpu.async_copy(w_hbm.at[cidx_v], c_v, sem)
  c1.wait()
  c2.wait()
  c3.wait()
  c4.wait()

  def step(i, _):
    sl = pl.ds(i * L, L)
    th = th_v[sl]
    a = a_v[sl]
    b = b_v[sl]
    c = c_v[sl]
    a_s = 1.0 / (1.0 + jnp.exp(-a))
    c_s = 1.0 / (1.0 + jnp.exp(-c))
    z = 1.0 / (1.0 + jnp.exp(-D_CONST * a_s * (th - b)))
    out_v[sl] = c_s + (1.0 - c_s) * z
    return 0

  lax.fori_loop(0, BPW // L, step, 0, unroll=4)

  pltpu.sync_copy(out_v, out_hbm.at[pl.ds(base, BPW)])


@jax.jit
def _irt_sc(user, item, theta_w, a_w, b_w, c_w):
  w = jnp.concatenate([
      jnp.squeeze(theta_w, -1), jnp.squeeze(a_w, -1),
      jnp.squeeze(b_w, -1), jnp.squeeze(c_w, -1)], axis=0)
  w = lax.optimization_barrier(w)
  uidx = user
  aidx = item + 1000000
  bidx = item + 2000000
  cidx = item + 3000000
  mesh = plsc.VectorSubcoreMesh(
      core_axis_name="c", subcore_axis_name="s",
      num_cores=NC, num_subcores=NS)
  fn = pl.kernel(
      _irt_body,
      out_type=jax.ShapeDtypeStruct((BATCH,), jnp.float32),
      mesh=mesh,
      scratch_types=[
          pltpu.VMEM((BPW,), jnp.int32),
          pltpu.VMEM((BPW,), jnp.int32),
          pltpu.VMEM((BPW,), jnp.int32),
          pltpu.VMEM((BPW,), jnp.int32),
          pltpu.VMEM((BPW,), jnp.float32),
          pltpu.VMEM((BPW,), jnp.float32),
          pltpu.VMEM((BPW,), jnp.float32),
          pltpu.VMEM((BPW,), jnp.float32),
          pltpu.VMEM((BPW,), jnp.float32),
          pltpu.SemaphoreType.DMA,
      ],
  )
  return fn(uidx, aidx, bidx, cidx, w)


def kernel(user, item, theta_w, a_w, b_w, c_w):
  return _irt_sc(user, item, theta_w, a_w, b_w, c_w)


# speedup vs baseline: 9.5583x; 2.1661x over previous
import functools

import jax
import jax.numpy as jnp
from jax import lax
from jax.experimental import pallas as pl
from jax.experimental.pallas import tpu as pltpu
from jax.experimental.pallas import tpu_sc as plsc

NC = 2
NS = 16
L = 16
BATCH = 16384
BPW = BATCH // (NC * NS)
D_CONST = 1.702


def _irt_body(user_hbm, item_hbm, theta_hbm, a_hbm, b_hbm, c_hbm, out_hbm,
              uidx_v, iidx_v, th_v, a_v, b_v, c_v, out_v, sem):
  wid = lax.axis_index("s") * NC + lax.axis_index("c")
  base = wid * BPW

  pltpu.sync_copy(user_hbm.at[pl.ds(base, BPW)], uidx_v)
  pltpu.sync_copy(item_hbm.at[pl.ds(base, BPW)], iidx_v)

  c1 = pltpu.async_copy(theta_hbm.at[uidx_v], th_v, sem)
  c2 = pltpu.async_copy(a_hbm.at[iidx_v], a_v, sem)
  c3 = pltpu.async_copy(b_hbm.at[iidx_v], b_v, sem)
  c4 = pltpu.async_copy(c_hbm.at[iidx_v], c_v, sem)
  c1.wait()
  c2.wait()
  c3.wait()
  c4.wait()

  def step(i, _):
    sl = pl.ds(i * L, L)
    th = th_v[sl]
    a = a_v[sl]
    b = b_v[sl]
    c = c_v[sl]
    a_s = 1.0 / (1.0 + jnp.exp(-a))
    c_s = 1.0 / (1.0 + jnp.exp(-c))
    z = 1.0 / (1.0 + jnp.exp(-D_CONST * a_s * (th - b)))
    out_v[sl] = c_s + (1.0 - c_s) * z
    return 0

  lax.fori_loop(0, BPW // L, step, 0, unroll=4)

  pltpu.sync_copy(out_v, out_hbm.at[pl.ds(base, BPW)])


@jax.jit
def _irt_sc(user, item, theta_w, a_w, b_w, c_w):
  w1 = jnp.squeeze(jnp.stack([theta_w, a_w], axis=0), -1)
  w2 = jnp.squeeze(jnp.stack([b_w, c_w], axis=0), -1)
  w1, w2 = lax.optimization_barrier((w1, w2))
  mesh = plsc.VectorSubcoreMesh(
      core_axis_name="c", subcore_axis_name="s",
      num_cores=NC, num_subcores=NS)
  fn = pl.kernel(
      _irt_body,
      out_type=jax.ShapeDtypeStruct((BATCH,), jnp.float32),
      mesh=mesh,
      scratch_types=[
          pltpu.VMEM((BPW,), jnp.int32),
          pltpu.VMEM((BPW,), jnp.int32),
          pltpu.VMEM((BPW,), jnp.float32),
          pltpu.VMEM((BPW,), jnp.float32),
          pltpu.VMEM((BPW,), jnp.float32),
          pltpu.VMEM((BPW,), jnp.float32),
          pltpu.VMEM((BPW,), jnp.float32),
          pltpu.SemaphoreType.DMA,
      ],
  )
  return fn(user, item, w1[0], w1[1], w2[0], w2[1])


def kernel(user, item, theta_w, a_w, b_w, c_w):
  return _irt_sc(user, item, theta_w, a_w, b_w, c_w)


# speedup vs baseline: 9.5703x; 1.0013x over previous
import jax
import jax.numpy as jnp
from jax import lax
from jax.experimental import pallas as pl
from jax.experimental.pallas import tpu as pltpu
from jax.experimental.pallas import tpu_sc as plsc

NC = 2
NS = 16
L = 16
BATCH = 16384
BPW = BATCH // (NC * NS)
D_CONST = 1.702


def _irt_body(user_hbm, item_hbm, theta_hbm, a_hbm, b_hbm, c_hbm, out_hbm,
              uidx_v, iidx_v, th_v, a_v, b_v, c_v, out_v, sem):
  wid = lax.axis_index("s") * NC + lax.axis_index("c")
  base = wid * BPW

  pltpu.sync_copy(user_hbm.at[pl.ds(base, BPW)], uidx_v)
  pltpu.sync_copy(item_hbm.at[pl.ds(base, BPW)], iidx_v)

  c1 = pltpu.async_copy(theta_hbm.at[uidx_v], th_v, sem)
  c2 = pltpu.async_copy(a_hbm.at[iidx_v], a_v, sem)
  c3 = pltpu.async_copy(b_hbm.at[iidx_v], b_v, sem)
  c4 = pltpu.async_copy(c_hbm.at[iidx_v], c_v, sem)
  c1.wait()
  c2.wait()
  c3.wait()
  c4.wait()

  def step(i, _):
    sl = pl.ds(i * L, L)
    th = th_v[sl]
    a = a_v[sl]
    b = b_v[sl]
    c = c_v[sl]
    a_s = 1.0 / (1.0 + jnp.exp(-a))
    c_s = 1.0 / (1.0 + jnp.exp(-c))
    z = 1.0 / (1.0 + jnp.exp(-D_CONST * a_s * (th - b)))
    out_v[sl] = c_s + (1.0 - c_s) * z
    return 0

  lax.fori_loop(0, BPW // L, step, 0, unroll=4)

  pltpu.sync_copy(out_v, out_hbm.at[pl.ds(base, BPW)])


@jax.jit
def _irt_sc(user, item, theta_w, a_w, b_w, c_w):
  w1 = jnp.squeeze(jnp.stack([theta_w, a_w], axis=0), -1)
  w2 = jnp.squeeze(jnp.stack([b_w, c_w], axis=0), -1)
  w1, w2 = lax.optimization_barrier((w1, w2))
  mesh = plsc.VectorSubcoreMesh(
      core_axis_name="c", subcore_axis_name="s",
      num_cores=NC, num_subcores=NS)
  fn = pl.kernel(
      _irt_body,
      out_type=jax.ShapeDtypeStruct((BATCH,), jnp.float32),
      mesh=mesh,
      scratch_types=[
          pltpu.VMEM((BPW,), jnp.int32),
          pltpu.VMEM((BPW,), jnp.int32),
          pltpu.VMEM((BPW,), jnp.float32),
          pltpu.VMEM((BPW,), jnp.float32),
          pltpu.VMEM((BPW,), jnp.float32),
          pltpu.VMEM((BPW,), jnp.float32),
          pltpu.VMEM((BPW,), jnp.float32),
          pltpu.SemaphoreType.DMA,
      ],
  )
  return fn(user, item, w1[0], w1[1], w2[0], w2[1])


def kernel(user, item, theta_w, a_w, b_w, c_w):
  return _irt_sc(user, item, theta_w, a_w, b_w, c_w)
